# baseline (speedup 1.0000x reference)
import functools

import jax
import jax.numpy as jnp
from jax import lax
from jax.experimental import pallas as pl
from jax.experimental.pallas import tpu as pltpu
from jax.experimental.pallas import tpu_sc as plsc

B = 16384
D_USER = 16
D_TS = 15
N_BOUNDS_PAD = 1024
TS_FLAT_PAD = 15024

_info = plsc.get_sparse_core_info()
NC = _info.num_cores
NS = _info.num_subcores
NW = NC * NS
BPW = B // NW
NGRP = BPW // 16
NSLOT = 6


def _fire(utabT_hbm, tile_v, slot, uid16, lanes, sem):
    for r in lanes:
        uid = uid16[r]
        blk = (uid // 128) * 128
        pltpu.async_copy(utabT_hbm.at[pl.ds(0, 16), pl.ds(blk, 128)],
                         tile_v.at[slot, r % 8], sem)


def _drain(utabT_hbm, tile_v, sem):
    for r in range(8):
        pltpu.make_async_copy(utabT_hbm.at[pl.ds(0, 16), pl.ds(0, 128)],
                              tile_v.at[0, r], sem).wait()


def _body(uid_hbm, ts_hbm, utabT_hbm, tsflat_hbm, bounds_hbm, mean_hbm,
          istd_hbm, out_hbm,
          idx_v, ts_v, bounds_v, tsflat_v, tile_v, out_v, scal_v, sem):
    wid = lax.axis_index("s") * NC + lax.axis_index("c")
    base = wid * BPW

    pltpu.sync_copy(uid_hbm.at[pl.ds(base, BPW)], idx_v)
    pltpu.sync_copy(ts_hbm.at[pl.ds(base, BPW)], ts_v)
    pltpu.sync_copy(bounds_hbm, bounds_v)
    pltpu.sync_copy(tsflat_hbm, tsflat_v)
    pltpu.sync_copy(mean_hbm, scal_v.at[0])
    pltpu.sync_copy(istd_hbm, scal_v.at[1])

    mean = scal_v[0, :]
    istd = scal_v[1, :]
    iota16 = lax.iota(jnp.int32, 16)
    zeros16 = jnp.zeros((16,), jnp.int32)

    uid0 = idx_v[pl.ds(0, 16)]
    uid1 = idx_v[pl.ds(16, 16)]
    uid2 = idx_v[pl.ds(32, 16)]
    _fire(utabT_hbm, tile_v, 0, uid0, range(8), sem)
    _fire(utabT_hbm, tile_v, 1, uid0, range(8, 16), sem)
    _fire(utabT_hbm, tile_v, 2, uid1, range(8), sem)
    _fire(utabT_hbm, tile_v, 3, uid1, range(8, 16), sem)
    _fire(utabT_hbm, tile_v, 4, uid2, range(8), sem)

    def group(g, carry):
        uid16 = idx_v[pl.ds(g * 16, 16)]
        s0 = lax.rem(2 * g, NSLOT)
        s1 = lax.rem(2 * g + 1, NSLOT)
        s5 = lax.rem(2 * g + 5, NSLOT)
        qcol = lax.rem(g, 8) * 16

        _drain(utabT_hbm, tile_v, sem)

        @pl.when(g + 2 < NGRP)
        def _():
            _fire(utabT_hbm, tile_v, s5, idx_v[pl.ds((g + 2) * 16, 16)],
                  range(8, 16), sem)

        ts16 = ts_v[pl.ds(g * 16, 16)]
        pos = zeros16
        step = N_BOUNDS_PAD // 2
        while step >= 1:
            probe = plsc.load_gather(bounds_v, [pos + (step - 1)])
            pos = jnp.where(probe <= ts16, pos + step, pos)
            step //= 2
        flat_base = pos * D_TS
        for j in range(D_TS):
            vals = plsc.load_gather(tsflat_v, [flat_base + j])
            out_v[D_USER + j, pl.ds(qcol, 16)] = vals
        norm = (ts16 - mean) * istd
        out_v[D_USER + D_TS, pl.ds(qcol, 16)] = norm

        for r in range(8):
            col = (uid16[r] - (uid16[r] // 128) * 128) + zeros16
            vals = plsc.load_gather(tile_v,
                                    [zeros16 + s0, zeros16 + r, iota16, col])
            plsc.store_scatter(out_v, [iota16, zeros16 + (qcol + r)], vals)

        _drain(utabT_hbm, tile_v, sem)

        @pl.when(g + 3 < NGRP)
        def _():
            _fire(utabT_hbm, tile_v, s0, idx_v[pl.ds((g + 3) * 16, 16)],
                  range(8), sem)

        for r in range(8, 16):
            col = (uid16[r] - (uid16[r] // 128) * 128) + zeros16
            vals = plsc.load_gather(tile_v,
                                    [zeros16 + s1, zeros16 + (r % 8), iota16,
                                     col])
            plsc.store_scatter(out_v, [iota16, zeros16 + (qcol + r)], vals)

        @pl.when(lax.rem(g, 8) == 7)
        def _():
            pltpu.sync_copy(out_v,
                            out_hbm.at[pl.ds(0, 2 * D_USER),
                                       pl.ds(base + (g // 8) * 128, 128)])
        return carry

    lax.fori_loop(0, NGRP, group, 0)


_sc_call = functools.partial(
    pl.kernel,
    out_type=jax.ShapeDtypeStruct((2 * D_USER, B), jnp.float32),
    mesh=plsc.VectorSubcoreMesh(core_axis_name="c", subcore_axis_name="s"),
    scratch_types=[
        pltpu.VMEM((BPW,), jnp.int32),
        pltpu.VMEM((BPW,), jnp.float32),
        pltpu.VMEM((N_BOUNDS_PAD,), jnp.float32),
        pltpu.VMEM((TS_FLAT_PAD,), jnp.float32),
        pltpu.VMEM((NSLOT, 8, 16, 128), jnp.float32),
        pltpu.VMEM((2 * D_USER, 128), jnp.float32),
        pltpu.VMEM((2, 16), jnp.float32),
        pltpu.SemaphoreType.DMA,
    ],
    compiler_params=pltpu.CompilerParams(needs_layout_passes=False),
)(_body)


def kernel(user_id, timestamp, user_table, ts_table, bin_boundaries, ts_mean,
           ts_var):
    uid32 = user_id.astype(jnp.int32)
    utabT = user_table.T
    bounds_pad = jnp.concatenate(
        [bin_boundaries,
         jnp.full((N_BOUNDS_PAD - bin_boundaries.shape[0],), jnp.inf,
                  jnp.float32)])
    tsflat = jnp.pad(ts_table.reshape(-1),
                     (0, TS_FLAT_PAD - ts_table.size))
    mean16 = jnp.full((16,), ts_mean, jnp.float32)
    istd16 = jnp.full((16,), lax.rsqrt(ts_var), jnp.float32)
    outT = _sc_call(uid32, timestamp, utabT, tsflat, bounds_pad, mean16,
                    istd16)
    return outT.T


# speedup vs baseline: 1.0370x; 1.0370x over previous
import functools

import jax
import jax.numpy as jnp
from jax import lax
from jax.experimental import pallas as pl
from jax.experimental.pallas import tpu as pltpu
from jax.experimental.pallas import tpu_sc as plsc

B = 16384
D_USER = 16
D_TS = 15
N_BOUNDS_PAD = 1024
TS_FLAT_PAD = 15024

_info = plsc.get_sparse_core_info()
NC = _info.num_cores
NS = _info.num_subcores
NW = NC * NS
BPW = B // NW
NGRP = BPW // 16
NSLOT = 5


def _fire(utabT_hbm, tile_v, slot, uid16, lanes, sem):
    for r in lanes:
        uid = uid16[r]
        blk = (uid // 128) * 128
        pltpu.async_copy(utabT_hbm.at[pl.ds(0, 16), pl.ds(blk, 128)],
                         tile_v.at[slot, r % 8], sem)


def _drain(utabT_hbm, tile_v, sem):
    for r in range(8):
        pltpu.make_async_copy(utabT_hbm.at[pl.ds(0, 16), pl.ds(0, 128)],
                              tile_v.at[0, r], sem).wait()


def _body(uid_hbm, ts_hbm, utabT_hbm, tsflat_hbm, bounds_hbm, mean_hbm,
          istd_hbm, out_hbm,
          idx_v, ts_v, bounds_v, tsflat_v, tile_v, out_v, scal_v, sem):
    wid = lax.axis_index("s") * NC + lax.axis_index("c")
    base = wid * BPW

    pltpu.sync_copy(uid_hbm.at[pl.ds(base, BPW)], idx_v)

    iota16 = lax.iota(jnp.int32, 16)
    zeros16 = jnp.zeros((16,), jnp.int32)

    uid0 = idx_v[pl.ds(0, 16)]
    uid1 = idx_v[pl.ds(16, 16)]
    _fire(utabT_hbm, tile_v, 0, uid0, range(8), sem)
    _fire(utabT_hbm, tile_v, 1, uid0, range(8, 16), sem)
    _fire(utabT_hbm, tile_v, 2, uid1, range(8), sem)
    _fire(utabT_hbm, tile_v, 3, uid1, range(8, 16), sem)

    pltpu.sync_copy(ts_hbm.at[pl.ds(base, BPW)], ts_v)
    pltpu.sync_copy(bounds_hbm, bounds_v)
    pltpu.sync_copy(tsflat_hbm, tsflat_v)
    pltpu.sync_copy(mean_hbm, scal_v.at[0])
    pltpu.sync_copy(istd_hbm, scal_v.at[1])

    mean = scal_v[0, :]
    istd = scal_v[1, :]

    def group(g, carry):
        uid16 = idx_v[pl.ds(g * 16, 16)]
        s0 = lax.rem(2 * g, NSLOT)
        s1 = lax.rem(2 * g + 1, NSLOT)
        s4 = lax.rem(2 * g + 4, NSLOT)
        s5 = lax.rem(2 * g + 5, NSLOT)

        _drain(utabT_hbm, tile_v, sem)

        @pl.when(g + 2 < NGRP)
        def _():
            _fire(utabT_hbm, tile_v, s4, idx_v[pl.ds((g + 2) * 16, 16)],
                  range(8), sem)

        ts16 = ts_v[pl.ds(g * 16, 16)]
        pos = zeros16
        step = N_BOUNDS_PAD // 2
        while step >= 1:
            probe = plsc.load_gather(bounds_v, [pos + (step - 1)])
            pos = jnp.where(probe <= ts16, pos + step, pos)
            step //= 2
        flat_base = pos * D_TS
        for j in range(D_TS):
            vals = plsc.load_gather(tsflat_v, [flat_base + j])
            out_v[D_USER + j, pl.ds(g * 16, 16)] = vals
        norm = (ts16 - mean) * istd
        out_v[D_USER + D_TS, pl.ds(g * 16, 16)] = norm

        for r in range(8):
            col = (uid16[r] - (uid16[r] // 128) * 128) + zeros16
            vals = plsc.load_gather(tile_v,
                                    [zeros16 + s0, zeros16 + r, iota16, col])
            plsc.store_scatter(out_v, [iota16, zeros16 + (g * 16 + r)], vals)

        _drain(utabT_hbm, tile_v, sem)

        @pl.when(g + 2 < NGRP)
        def _():
            _fire(utabT_hbm, tile_v, s5, idx_v[pl.ds((g + 2) * 16, 16)],
                  range(8, 16), sem)

        for r in range(8, 16):
            col = (uid16[r] - (uid16[r] // 128) * 128) + zeros16
            vals = plsc.load_gather(tile_v,
                                    [zeros16 + s1, zeros16 + (r % 8), iota16,
                                     col])
            plsc.store_scatter(out_v, [iota16, zeros16 + (g * 16 + r)], vals)
        return carry

    lax.fori_loop(0, NGRP, group, 0)
    pltpu.sync_copy(out_v, out_hbm.at[pl.ds(0, 2 * D_USER), pl.ds(base, BPW)])


_sc_call = functools.partial(
    pl.kernel,
    out_type=jax.ShapeDtypeStruct((2 * D_USER, B), jnp.float32),
    mesh=plsc.VectorSubcoreMesh(core_axis_name="c", subcore_axis_name="s"),
    scratch_types=[
        pltpu.VMEM((BPW,), jnp.int32),
        pltpu.VMEM((BPW,), jnp.float32),
        pltpu.VMEM((N_BOUNDS_PAD,), jnp.float32),
        pltpu.VMEM((TS_FLAT_PAD,), jnp.float32),
        pltpu.VMEM((NSLOT, 8, 16, 128), jnp.float32),
        pltpu.VMEM((2 * D_USER, BPW), jnp.float32),
        pltpu.VMEM((2, 16), jnp.float32),
        pltpu.SemaphoreType.DMA,
    ],
    compiler_params=pltpu.CompilerParams(needs_layout_passes=False),
)(_body)


def kernel(user_id, timestamp, user_table, ts_table, bin_boundaries, ts_mean,
           ts_var):
    uid32 = user_id.astype(jnp.int32)
    utabT = user_table.T
    bounds_pad = jnp.concatenate(
        [bin_boundaries,
         jnp.full((N_BOUNDS_PAD - bin_boundaries.shape[0],), jnp.inf,
                  jnp.float32)])
    tsflat = jnp.pad(ts_table.reshape(-1),
                     (0, TS_FLAT_PAD - ts_table.size))
    mean16 = jnp.full((16,), ts_mean, jnp.float32)
    istd16 = jnp.full((16,), lax.rsqrt(ts_var), jnp.float32)
    outT = _sc_call(uid32, timestamp, utabT, tsflat, bounds_pad, mean16,
                    istd16)
    return outT.T
